# baseline (speedup 1.0000x reference)
import functools

import jax
import jax.numpy as jnp
from jax import lax
from jax.experimental import pallas as pl
from jax.experimental.pallas import tpu as pltpu
from jax.experimental.pallas import tpu_sc as plsc


def _sc_denom_body(nrows, ncols, adj_hbm, out_hbm, bufs, dstage, sem0, sem1):
    nw = 32
    rpw = nrows // nw
    ng = rpw // 16
    ch = 16 * ncols
    wid = lax.axis_index("s") * 2 + lax.axis_index("c")
    base = wid * rpw
    lanes = lax.iota(jnp.int32, 16)

    def start(g, buf, sem):
        pltpu.make_async_copy(
            adj_hbm.at[pl.ds((base + g * 16) * ncols, ch)], buf, sem
        ).start()

    def wait(buf, sem):
        pltpu.make_async_copy(
            adj_hbm.at[pl.ds(0, ch)], buf, sem
        ).wait()

    def scan_group(g, bsel, sem, nxt):
        wait(bufs.at[bsel], sem)

        @pl.when(nxt < ng)
        def _():
            start(nxt, bufs.at[bsel], sem)

        def row(r, carry):
            off = r * ncols

            def col(t, accs):
                a0, a1, a2, a3 = accs
                p = off + t * 64
                a0 = a0 + bufs[bsel, pl.ds(p, 16)]
                a1 = a1 + bufs[bsel, pl.ds(p + 16, 16)]
                a2 = a2 + bufs[bsel, pl.ds(p + 32, 16)]
                a3 = a3 + bufs[bsel, pl.ds(p + 48, 16)]
                return (a0, a1, a2, a3)

            z = jnp.zeros((16,), jnp.float32)
            a0, a1, a2, a3 = lax.fori_loop(0, ncols // 64, col, (z, z, z, z))
            av = (a0 + a1) + (a2 + a3)
            s = jnp.float32(1.0)
            for l in range(16):
                s = s + av[l]
            return jnp.where(lanes == r, s, carry)

        gs = lax.fori_loop(0, 16, row, jnp.zeros((16,), jnp.float32))
        dstage[pl.ds(g * 16, 16)] = gs

    start(0, bufs.at[0], sem0)
    start(1, bufs.at[1], sem1)

    def body(gg, carry):
        g0 = gg * 2
        scan_group(g0, 0, sem0, g0 + 2)
        scan_group(g0 + 1, 1, sem1, g0 + 3)
        return carry

    lax.fori_loop(0, ng // 2, body, 0)
    pltpu.sync_copy(dstage, out_hbm.at[pl.ds(base, rpw)])


def _sc_denoms(adj):
    B, N, _ = adj.shape
    mesh = plsc.VectorSubcoreMesh(core_axis_name="c", subcore_axis_name="s")
    return pl.kernel(
        functools.partial(_sc_denom_body, B * N, N),
        out_type=jax.ShapeDtypeStruct((B * N,), jnp.float32),
        mesh=mesh,
        scratch_types=[
            pltpu.VMEM((2, 16 * N), jnp.float32),
            pltpu.VMEM(((B * N) // 32,), jnp.float32),
            pltpu.SemaphoreType.DMA,
            pltpu.SemaphoreType.DMA,
        ],
    )(adj.reshape(-1))


def _gcn_body(bn, nh, adjl_ref, adjr_ref, nodes_all_ref, w0_ref, b0_ref,
              wo_ref, bo_ref, out_ref):
    i = pl.program_id(1)
    al = adjl_ref[0]
    ar = adjr_ref[0]
    denom = (
        jnp.sum(al, axis=1, keepdims=True)
        + jnp.sum(ar, axis=1, keepdims=True)
        + 1.0
    )
    nodes_blk = nodes_all_ref[0, pl.ds(i * bn, bn), :]
    nl = nodes_all_ref[0, pl.ds(0, nh), :].astype(jnp.bfloat16)
    nr = nodes_all_ref[0, pl.ds(nh, nh), :].astype(jnp.bfloat16)
    h = (
        jnp.dot(al.astype(jnp.bfloat16), nl, preferred_element_type=jnp.float32)
        + jnp.dot(ar.astype(jnp.bfloat16), nr, preferred_element_type=jnp.float32)
        + nodes_blk
    )
    hw = jnp.dot(h, w0_ref[...], preferred_element_type=jnp.float32)
    g = jnp.maximum(hw / denom + b0_ref[0], 0.0) + nodes_blk
    out_ref[0] = (
        jnp.dot(g, wo_ref[...], preferred_element_type=jnp.float32) + bo_ref[0]
    )


def kernel(nodes, adj, W0, b0, Wo, bo):
    B, N, D = nodes.shape
    BN = 1024
    NH = N // 2

    den = _sc_denoms(adj)

    out = pl.pallas_call(
        functools.partial(_gcn_body, BN, NH),
        grid=(B, N // BN),
        in_specs=[
            pl.BlockSpec((1, BN, NH), lambda b, i: (b, i, 0)),
            pl.BlockSpec((1, BN, NH), lambda b, i: (b, i, 1)),
            pl.BlockSpec((1, N, D), lambda b, i: (b, 0, 0)),
            pl.BlockSpec((D, D), lambda b, i: (0, 0)),
            pl.BlockSpec((1, D), lambda b, i: (0, 0)),
            pl.BlockSpec((D, D), lambda b, i: (0, 0)),
            pl.BlockSpec((1, D), lambda b, i: (0, 0)),
        ],
        out_specs=pl.BlockSpec((1, BN, D), lambda b, i: (b, i, 0)),
        out_shape=jax.ShapeDtypeStruct((B, N, D), jnp.float32),
        compiler_params=pltpu.CompilerParams(
            dimension_semantics=("parallel", "parallel"),
        ),
    )(adj, adj, nodes, W0, b0.reshape(1, D), Wo, bo.reshape(1, D))
    return out + jnp.minimum(jnp.min(den), 0.0)


# speedup vs baseline: 4.9394x; 4.9394x over previous
import functools

import jax
import jax.numpy as jnp
from jax.experimental import pallas as pl
from jax.experimental.pallas import tpu as pltpu


def _gcn_body(bn, nh, adjl_ref, adjr_ref, nodes_all_ref, w0_ref, b0_ref,
              wo_ref, bo_ref, out_ref):
    i = pl.program_id(1)
    al = adjl_ref[0]
    ar = adjr_ref[0]
    denom = (
        jnp.sum(al, axis=1, keepdims=True)
        + jnp.sum(ar, axis=1, keepdims=True)
        + 1.0
    )
    nodes_blk = nodes_all_ref[0, pl.ds(i * bn, bn), :]
    nl = nodes_all_ref[0, pl.ds(0, nh), :].astype(jnp.bfloat16)
    nr = nodes_all_ref[0, pl.ds(nh, nh), :].astype(jnp.bfloat16)
    h = (
        jnp.dot(al.astype(jnp.bfloat16), nl, preferred_element_type=jnp.float32)
        + jnp.dot(ar.astype(jnp.bfloat16), nr, preferred_element_type=jnp.float32)
        + nodes_blk
    )
    hw = jnp.dot(h, w0_ref[...], preferred_element_type=jnp.float32)
    g = jnp.maximum(hw / denom + b0_ref[0], 0.0) + nodes_blk
    out_ref[0] = (
        jnp.dot(g, wo_ref[...], preferred_element_type=jnp.float32) + bo_ref[0]
    )


def kernel(nodes, adj, W0, b0, Wo, bo):
    B, N, D = nodes.shape
    BN = 1024
    NH = N // 2

    return pl.pallas_call(
        functools.partial(_gcn_body, BN, NH),
        grid=(B, N // BN),
        in_specs=[
            pl.BlockSpec((1, BN, NH), lambda b, i: (b, i, 0)),
            pl.BlockSpec((1, BN, NH), lambda b, i: (b, i, 1)),
            pl.BlockSpec((1, N, D), lambda b, i: (b, 0, 0)),
            pl.BlockSpec((D, D), lambda b, i: (0, 0)),
            pl.BlockSpec((1, D), lambda b, i: (0, 0)),
            pl.BlockSpec((D, D), lambda b, i: (0, 0)),
            pl.BlockSpec((1, D), lambda b, i: (0, 0)),
        ],
        out_specs=pl.BlockSpec((1, BN, D), lambda b, i: (b, i, 0)),
        out_shape=jax.ShapeDtypeStruct((B, N, D), jnp.float32),
        compiler_params=pltpu.CompilerParams(
            dimension_semantics=("parallel", "parallel"),
        ),
    )(adj, adj, nodes, W0, b0.reshape(1, D), Wo, bo.reshape(1, D))
